# baseline (speedup 1.0000x reference)
import functools

import jax
import jax.numpy as jnp
from jax import lax
from jax.experimental import pallas as pl
from jax.experimental.pallas import tpu as pltpu
from jax.experimental.pallas import tpu_sc as plsc

N_NODES = 10000
N_EDGES = 160000
NP = 10240
EP = 174080
E_T = EP // 16
_B = 64
_NB2 = E_T // _B
NB = E_T // 128
N_STRIPE = NP // 16

_C = tuple((16 * (p // 32) + (p % 16) + 64 * ((p % 32) // 16))
           for p in range(128))

_MESH = dict(core_axis_name="c", subcore_axis_name="s")


def _newton_rsqrt(v):
    i = lax.bitcast_convert_type(v, jnp.int32)
    i = jnp.int32(0x5F3759DF) - lax.shift_right_arithmetic(i, 1)
    y = lax.bitcast_convert_type(i, jnp.float32)
    for _ in range(3):
        y = y * (1.5 - 0.5 * v * y * y)
    return y


@functools.partial(
    pl.kernel,
    mesh=plsc.VectorSubcoreMesh(**_MESH),
    out_type=jax.ShapeDtypeStruct((NP,), jnp.float32),
    scratch_types=[
        pltpu.VMEM((NB, 128), jnp.int32),
        pltpu.VMEM((NB, 128), jnp.float32),
        pltpu.VMEM((N_STRIPE,), jnp.float32),
        pltpu.VMEM_SHARED((NP,), jnp.float32),
        pltpu.SemaphoreType.DMA,
    ],
)
def _dinv_kernel(dst_hbm, ew_hbm, out_hbm, dstv, ewv, nodev, deg_sh, sem):
    cid = lax.axis_index("c")
    sid = lax.axis_index("s")

    def _zero(i, _):
        nodev[pl.ds(i * 16, 16)] = jnp.zeros((16,), jnp.float32)
        return _
    lax.fori_loop(0, N_STRIPE // 16, _zero, None)
    pltpu.sync_copy(nodev, deg_sh.at[pl.ds(sid * N_STRIPE, N_STRIPE)])
    plsc.subcore_barrier()

    pltpu.sync_copy(dst_hbm.at[sid], dstv)
    pltpu.sync_copy(ew_hbm.at[sid], ewv)

    def _scat(j, _):
        pltpu.sync_copy(ewv.at[j], deg_sh.at[dstv.at[j]], add=True)
        return _
    lax.fori_loop(0, NB, _scat, None)
    plsc.subcore_barrier()

    n_t = NP // 32
    g0 = cid * (NP // 2) + sid * n_t
    pltpu.sync_copy(deg_sh.at[pl.ds(g0, n_t)], nodev.at[pl.ds(0, n_t)])

    def _rs(i, _):
        v = jnp.maximum(nodev[pl.ds(i * 16, 16)], 1.0)
        nodev[pl.ds(i * 16, 16)] = _newton_rsqrt(v)
        return _
    lax.fori_loop(0, n_t // 16, _rs, None)
    pltpu.sync_copy(nodev.at[pl.ds(0, n_t)], out_hbm.at[pl.ds(g0, n_t)])


def _make_prop(wc):
    cps = wc // 2

    @functools.partial(
        pl.kernel,
        mesh=plsc.VectorSubcoreMesh(**_MESH),
        out_type=jax.ShapeDtypeStruct((wc, NP, 128), jnp.float32),
        compiler_params=pltpu.CompilerParams(
            needs_layout_passes=False, use_tc_tiling_on_sc=False),
        scratch_types=[
            pltpu.VMEM((_B,), jnp.float32),
            pltpu.VMEM((_B,), jnp.float32),
            pltpu.VMEM((_NB2, _B), jnp.int32),
            pltpu.VMEM((E_T,), jnp.int32),
            pltpu.VMEM((_B, 64), jnp.int32),
            pltpu.VMEM((_B, 64), jnp.int32),
            pltpu.VMEM((_B, 128), jnp.float32),
            pltpu.VMEM((_B, 128), jnp.float32),
            pltpu.VMEM_SHARED((NP, 128), jnp.float32),
            pltpu.SemaphoreType.DMA,
            pltpu.SemaphoreType.DMA,
            pltpu.SemaphoreType.DMA,
            pltpu.SemaphoreType.DMA,
            pltpu.SemaphoreType.DMA,
            pltpu.SemaphoreType.DMA,
        ],
    )
    def _prop(src_hbm, dst_hbm, ew_hbm, tbl_hbm, out_hbm,
              ewb0, ewb1, dstv, idxv, rb0, rb1,
              rf0, rf1, agg_sh,
              semg0, semg1, sems0, sems1, seme0, seme1):
        cid = lax.axis_index("c")
        sid = lax.axis_index("s")
        e0 = sid * E_T
        pltpu.sync_copy(src_hbm.at[pl.ds(e0, E_T)], idxv)
        pltpu.sync_copy(dst_hbm.at[sid], dstv)

        rowsb = (rb0, rb1)
        rowsf = (rf0, rf1)
        ewb = (ewb0, ewb1)
        semg = (semg0, semg1)
        sems = (sems0, sems1)
        seme = (seme0, seme1)

        def _gather(b, buf):
            pltpu.async_copy(
                ew_hbm.at[pl.ds(e0 + b * _B, _B)], ewb[buf], seme[buf])
            pltpu.async_copy(
                tbl_hbm.at[idxv.at[pl.ds(b * _B, _B)]], rowsb[buf], semg[buf])

        def _gather_wait(b, buf):
            pltpu.make_async_copy(
                tbl_hbm.at[idxv.at[pl.ds(b * _B, _B)]], rowsb[buf], semg[buf]
            ).wait()

        def _scat(b, buf):
            pltpu.async_copy(
                rowsf[buf], agg_sh.at[dstv.at[b]], sems[buf], add=True)

        def _scat_wait(b, buf):
            pltpu.make_async_copy(
                rowsf[buf], agg_sh.at[dstv.at[b]], sems[buf]).wait()

        def _scale(b, gbuf, fbuf):
            pltpu.make_async_copy(
                ew_hbm.at[pl.ds(e0 + b * _B, _B)], ewb[gbuf], seme[gbuf]
            ).wait()
            mask = jnp.full((16,), -65536, jnp.int32)

            def _sg(g, _):
                wv = ewb[gbuf][pl.ds(g * 16, 16)]
                for e16 in range(16):
                    w = wv[e16]
                    r = g * 16 + e16
                    for k in range(4):
                        mi = rowsb[gbuf][r, pl.ds(k * 16, 16)]
                        lo = lax.bitcast_convert_type(
                            lax.shift_left(mi, 16), jnp.float32)
                        hi = lax.bitcast_convert_type(mi & mask, jnp.float32)
                        rowsf[fbuf][r, pl.ds(k * 32, 16)] = lo * w
                        rowsf[fbuf][r, pl.ds(k * 32 + 16, 16)] = hi * w
                return _
            lax.fori_loop(0, _B // 16, _sg, None)

        for fci in range(cps):
            fcg = cid * cps + fci
            def _zb(j, _):
                for k in range(8):
                    rf0[j, pl.ds(k * 16, 16)] = jnp.zeros((16,), jnp.float32)
                return _
            lax.fori_loop(0, _B, _zb, None)
            for t in range(N_STRIPE // _B):
                pltpu.sync_copy(
                    rf0, agg_sh.at[pl.ds(sid * N_STRIPE + t * _B, _B)])
            delta = cid * cps * NP if fci == 0 else NP

            def _idx(i, _):
                idxv[pl.ds(i * 16, 16)] = idxv[pl.ds(i * 16, 16)] + delta
                return _
            lax.fori_loop(0, E_T // 16, _idx, None)
            plsc.subcore_barrier()

            _gather(0, 0)

            def _pair(b2, _):
                b = 2 * b2

                @pl.when(b2 > 0)
                def _():
                    _scat_wait(b - 1, 1)
                _gather(b + 1, 1)
                _gather_wait(b, 0)
                _scale(b, 0, 0)
                _scat(b, 0)

                _scat_wait(b, 0)

                @pl.when(b2 < _NB2 // 2 - 1)
                def _():
                    _gather(b + 2, 0)
                _gather_wait(b + 1, 1)
                _scale(b + 1, 1, 1)
                _scat(b + 1, 1)
                return _
            lax.fori_loop(0, _NB2 // 2, _pair, None)
            _scat_wait(_NB2 - 1, 1)
            plsc.subcore_barrier()
            pltpu.sync_copy(
                agg_sh.at[pl.ds(sid * N_STRIPE, N_STRIPE)],
                out_hbm.at[fcg, pl.ds(sid * N_STRIPE, N_STRIPE)])
            plsc.subcore_barrier()

    return _prop


_prop4 = _make_prop(4)
_prop2 = _make_prop(2)


_DOT = dict(precision=lax.Precision.HIGHEST, preferred_element_type=jnp.float32)
_RB = 1024


def _pack_bf16_pairs(h):
    bi = lax.bitcast_convert_type(h, jnp.int32) + jnp.int32(0x8000)
    word = (bi[:, 64:] & jnp.int32(-65536)) | lax.shift_right_logical(
        bi[:, :64], 16)
    return word


def _tc1(xp, w1, dinv2):
    def body(x_ref, w_ref, dv_ref, o_ref):
        h = jnp.dot(x_ref[...] * dv_ref[...], w_ref[...], **_DOT)
        o_ref[0] = _pack_bf16_pairs(h)

    return pl.pallas_call(
        body,
        grid=(NP // _RB, 4),
        in_specs=[
            pl.BlockSpec((_RB, 256), lambda i, j: (i, 0)),
            pl.BlockSpec((256, 128), lambda i, j: (0, j)),
            pl.BlockSpec((_RB, 1), lambda i, j: (i, 0)),
        ],
        out_specs=pl.BlockSpec((1, _RB, 64), lambda i, j: (j, i, 0)),
        out_shape=jax.ShapeDtypeStruct((4, NP, 64), jnp.int32),
    )(xp, w1, dinv2)


def _tc2(agg1, dinv2, b1c, w2c):
    def body(a_ref, dv_ref, b_ref, w_ref, o_ref):
        dv = dv_ref[...][None]
        t = jnp.maximum(dv * a_ref[...] + b_ref[...][:, None, :], 0.0)
        acc = jnp.zeros((_RB, 128), jnp.float32)
        for c in range(4):
            acc = acc + jnp.dot(t[c], w_ref[0, c], **_DOT)
        o_ref[0] = _pack_bf16_pairs(dv_ref[...] * acc)

    return pl.pallas_call(
        body,
        grid=(NP // _RB, 2),
        in_specs=[
            pl.BlockSpec((4, _RB, 128), lambda i, o: (0, i, 0)),
            pl.BlockSpec((_RB, 1), lambda i, o: (i, 0)),
            pl.BlockSpec((4, 128), lambda i, o: (0, 0)),
            pl.BlockSpec((1, 4, 128, 128), lambda i, o: (o, 0, 0, 0)),
        ],
        out_specs=pl.BlockSpec((1, _RB, 64), lambda i, o: (o, i, 0)),
        out_shape=jax.ShapeDtypeStruct((2, NP, 64), jnp.int32),
    )(agg1, dinv2, b1c, w2c)


def _tc3(agg2, dinv2, b2c, l1c, bl1, l2, bl2, l3, bl3):
    def body(a_ref, dv_ref, b_ref, l1_ref, c1_ref, l2_ref, c2_ref,
             l3_ref, c3_ref, o_ref):
        dv = dv_ref[...][None]
        z = jnp.maximum(dv * a_ref[...] + b_ref[...][:, None, :], 0.0)
        m1 = jnp.dot(z[0], l1_ref[0], **_DOT) + jnp.dot(z[1], l1_ref[1], **_DOT)
        m1 = jnp.maximum(m1 + c1_ref[...], 0.0)
        m2 = jnp.maximum(jnp.dot(m1, l2_ref[...], **_DOT) + c2_ref[...], 0.0)
        lg = jnp.dot(m2, l3_ref[...], **_DOT) + c3_ref[...]
        lg = lg - jnp.max(lg, axis=-1, keepdims=True)
        e = jnp.exp(lg)
        o_ref[...] = e / jnp.sum(e, axis=-1, keepdims=True)

    return pl.pallas_call(
        body,
        grid=(NP // _RB,),
        in_specs=[
            pl.BlockSpec((2, _RB, 128), lambda i: (0, i, 0)),
            pl.BlockSpec((_RB, 1), lambda i: (i, 0)),
            pl.BlockSpec((2, 128), lambda i: (0, 0)),
            pl.BlockSpec((2, 128, 128), lambda i: (0, 0, 0)),
            pl.BlockSpec((1, 128), lambda i: (0, 0)),
            pl.BlockSpec((128, 64), lambda i: (0, 0)),
            pl.BlockSpec((1, 64), lambda i: (0, 0)),
            pl.BlockSpec((64, 40), lambda i: (0, 0)),
            pl.BlockSpec((1, 40), lambda i: (0, 0)),
        ],
        out_specs=pl.BlockSpec((_RB, 40), lambda i: (i, 0)),
        out_shape=jax.ShapeDtypeStruct((NP, 40), jnp.float32),
    )(agg2, dinv2, b2c, l1c, bl1, l2, bl2, l3, bl3)


def kernel(x, edge_index, edge_weights, W1, b1, W2, b2, L1, bl1, L2, bl2,
           L3, bl3):
    src = edge_index[0].astype(jnp.int32)
    dst = edge_index[1].astype(jnp.int32)
    ew = edge_weights.astype(jnp.float32)
    ci = jnp.asarray(_C, jnp.int32)

    pe = EP - N_EDGES - N_NODES
    loop = jnp.arange(N_NODES, dtype=jnp.int32)
    pad_ids = jnp.arange(pe, dtype=jnp.int32)
    src_p = jnp.concatenate([src, loop, pad_ids % N_NODES])
    dst_p = jnp.concatenate([dst, loop, N_NODES + pad_ids % (NP - N_NODES)])
    ew_p = jnp.concatenate([ew, jnp.ones((N_NODES,), jnp.float32),
                            jnp.zeros((pe,), jnp.float32)])
    dst2d = dst_p.reshape(16, NB, 128)
    dstb = dst_p.reshape(16, _NB2, _B)
    ew2d = ew_p.reshape(16, NB, 128)

    xp = jnp.pad(x.astype(jnp.float32), ((0, NP - N_NODES), (0, 0)))
    dinv = _dinv_kernel(dst2d, ew2d)
    dinv2 = dinv.reshape(NP, 1)

    b1c = b1.reshape(4, 128)[:, ci]
    b2c = b2.reshape(2, 128)[:, ci]
    w2c = W2.reshape(4, 128, 2, 128)[:, ci].transpose(2, 0, 1, 3)
    l1c = L1.reshape(2, 128, 128)[:, ci]

    h1b = _tc1(xp, W1, dinv2)
    agg1 = _prop4(src_p, dstb, ew_p, h1b.reshape(4 * NP, 64))
    h2b = _tc2(agg1, dinv2, b1c, w2c)
    agg2 = _prop2(src_p, dstb, ew_p, h2b.reshape(2 * NP, 64))
    out = _tc3(agg2, dinv2, b2c, l1c, bl1.reshape(1, 128),
               L2, bl2.reshape(1, 64), L3, bl3.reshape(1, 40))
    return out[:N_NODES]


# speedup vs baseline: 1.8514x; 1.8514x over previous
import functools

import jax
import jax.numpy as jnp
from jax import lax
from jax.experimental import pallas as pl
from jax.experimental.pallas import tpu as pltpu
from jax.experimental.pallas import tpu_sc as plsc

N_NODES = 10000
N_EDGES = 160000
NP = 10240
EP = 174080
E_T = EP // 16
_B = 64
_NB2 = E_T // _B
NB = E_T // 128
N_STRIPE = NP // 16

_C = tuple((16 * (p // 32) + (p % 16) + 64 * ((p % 32) // 16))
           for p in range(128))

_MESH = dict(core_axis_name="c", subcore_axis_name="s")


def _newton_rsqrt(v):
    i = lax.bitcast_convert_type(v, jnp.int32)
    i = jnp.int32(0x5F3759DF) - lax.shift_right_arithmetic(i, 1)
    y = lax.bitcast_convert_type(i, jnp.float32)
    for _ in range(3):
        y = y * (1.5 - 0.5 * v * y * y)
    return y


@functools.partial(
    pl.kernel,
    mesh=plsc.VectorSubcoreMesh(**_MESH),
    out_type=jax.ShapeDtypeStruct((NP,), jnp.float32),
    scratch_types=[
        pltpu.VMEM((NB, 128), jnp.int32),
        pltpu.VMEM((NB, 128), jnp.float32),
        pltpu.VMEM((N_STRIPE,), jnp.float32),
        pltpu.VMEM_SHARED((NP,), jnp.float32),
        pltpu.SemaphoreType.DMA,
    ],
)
def _dinv_kernel(dst_hbm, ew_hbm, out_hbm, dstv, ewv, nodev, deg_sh, sem):
    cid = lax.axis_index("c")
    sid = lax.axis_index("s")

    def _zero(i, _):
        nodev[pl.ds(i * 16, 16)] = jnp.zeros((16,), jnp.float32)
        return _
    lax.fori_loop(0, N_STRIPE // 16, _zero, None)
    pltpu.sync_copy(nodev, deg_sh.at[pl.ds(sid * N_STRIPE, N_STRIPE)])
    plsc.subcore_barrier()

    pltpu.sync_copy(dst_hbm.at[sid], dstv)
    pltpu.sync_copy(ew_hbm.at[sid], ewv)

    def _scat(j, _):
        pltpu.sync_copy(ewv.at[j], deg_sh.at[dstv.at[j]], add=True)
        return _
    lax.fori_loop(0, NB, _scat, None)
    plsc.subcore_barrier()

    n_t = NP // 32
    g0 = cid * (NP // 2) + sid * n_t
    pltpu.sync_copy(deg_sh.at[pl.ds(g0, n_t)], nodev.at[pl.ds(0, n_t)])

    def _rs(i, _):
        v = jnp.maximum(nodev[pl.ds(i * 16, 16)], 1.0)
        nodev[pl.ds(i * 16, 16)] = _newton_rsqrt(v)
        return _
    lax.fori_loop(0, n_t // 16, _rs, None)
    pltpu.sync_copy(nodev.at[pl.ds(0, n_t)], out_hbm.at[pl.ds(g0, n_t)])


def _make_prop(wc):
    cps = wc // 2

    @functools.partial(
        pl.kernel,
        mesh=plsc.VectorSubcoreMesh(**_MESH),
        out_type=jax.ShapeDtypeStruct((wc, NP, 128), jnp.float32),
        compiler_params=pltpu.CompilerParams(
            needs_layout_passes=False, use_tc_tiling_on_sc=False),
        scratch_types=[
            pltpu.VMEM((_B,), jnp.float32),
            pltpu.VMEM((_B,), jnp.float32),
            pltpu.VMEM((_NB2, _B), jnp.int32),
            pltpu.VMEM((E_T,), jnp.int32),
            pltpu.VMEM((_B, 64), jnp.int32),
            pltpu.VMEM((_B, 64), jnp.int32),
            pltpu.VMEM((_B, 128), jnp.float32),
            pltpu.VMEM((_B, 128), jnp.float32),
            pltpu.VMEM_SHARED((NP, 128), jnp.float32),
            pltpu.SemaphoreType.DMA,
            pltpu.SemaphoreType.DMA,
            pltpu.SemaphoreType.DMA,
            pltpu.SemaphoreType.DMA,
            pltpu.SemaphoreType.DMA,
            pltpu.SemaphoreType.DMA,
        ],
    )
    def _prop(src_hbm, dst_hbm, ew_hbm, tbl_hbm, out_hbm,
              ewb0, ewb1, dstv, idxv, rb0, rb1,
              rf0, rf1, agg_sh,
              semg0, semg1, sems0, sems1, seme0, seme1):
        cid = lax.axis_index("c")
        sid = lax.axis_index("s")
        e0 = sid * E_T
        pltpu.sync_copy(src_hbm.at[pl.ds(e0, E_T)], idxv)
        pltpu.sync_copy(dst_hbm.at[sid], dstv)

        rowsb = (rb0, rb1)
        rowsf = (rf0, rf1)
        ewb = (ewb0, ewb1)
        semg = (semg0, semg1)
        sems = (sems0, sems1)
        seme = (seme0, seme1)

        def _gather(b, buf):
            pltpu.async_copy(
                ew_hbm.at[pl.ds(e0 + b * _B, _B)], ewb[buf], seme[buf])
            pltpu.async_copy(
                tbl_hbm.at[idxv.at[pl.ds(b * _B, _B)]], rowsb[buf], semg[buf])

        def _gather_wait(b, buf):
            pltpu.make_async_copy(
                tbl_hbm.at[idxv.at[pl.ds(b * _B, _B)]], rowsb[buf], semg[buf]
            ).wait()

        def _scat(b, buf):
            pltpu.async_copy(
                rowsf[buf], agg_sh.at[dstv.at[b]], sems[buf], add=True)

        def _scat_wait(b, buf):
            pltpu.make_async_copy(
                rowsf[buf], agg_sh.at[dstv.at[b]], sems[buf]).wait()

        def _scale(b, gbuf, fbuf):
            pltpu.make_async_copy(
                ew_hbm.at[pl.ds(e0 + b * _B, _B)], ewb[gbuf], seme[gbuf]
            ).wait()
            mask = jnp.full((16,), -65536, jnp.int32)
            for g in range(_B // 16):
                wv = ewb[gbuf][pl.ds(g * 16, 16)]
                for e16 in range(16):
                    w = wv[e16]
                    r = g * 16 + e16
                    for k in range(4):
                        mi = rowsb[gbuf][r, pl.ds(k * 16, 16)]
                        lo = lax.bitcast_convert_type(
                            lax.shift_left(mi, 16), jnp.float32)
                        hi = lax.bitcast_convert_type(mi & mask, jnp.float32)
                        rowsf[fbuf][r, pl.ds(k * 32, 16)] = lo * w
                        rowsf[fbuf][r, pl.ds(k * 32 + 16, 16)] = hi * w

        for fci in range(cps):
            fcg = cid * cps + fci
            def _zb(j, _):
                for k in range(8):
                    rf0[j, pl.ds(k * 16, 16)] = jnp.zeros((16,), jnp.float32)
                return _
            lax.fori_loop(0, _B, _zb, None)
            for t in range(N_STRIPE // _B):
                pltpu.sync_copy(
                    rf0, agg_sh.at[pl.ds(sid * N_STRIPE + t * _B, _B)])
            delta = cid * cps * NP if fci == 0 else NP

            def _idx(i, _):
                idxv[pl.ds(i * 16, 16)] = idxv[pl.ds(i * 16, 16)] + delta
                return _
            lax.fori_loop(0, E_T // 16, _idx, None)
            plsc.subcore_barrier()

            _gather(0, 0)

            def _pair(b2, _):
                b = 2 * b2

                @pl.when(b2 > 0)
                def _():
                    _scat_wait(b - 1, 1)
                _gather(b + 1, 1)
                _gather_wait(b, 0)
                _scale(b, 0, 0)
                _scat(b, 0)

                _scat_wait(b, 0)

                @pl.when(b2 < _NB2 // 2 - 1)
                def _():
                    _gather(b + 2, 0)
                _gather_wait(b + 1, 1)
                _scale(b + 1, 1, 1)
                _scat(b + 1, 1)
                return _
            lax.fori_loop(0, _NB2 // 2, _pair, None)
            _scat_wait(_NB2 - 1, 1)
            plsc.subcore_barrier()
            pltpu.sync_copy(
                agg_sh.at[pl.ds(sid * N_STRIPE, N_STRIPE)],
                out_hbm.at[fcg, pl.ds(sid * N_STRIPE, N_STRIPE)])
            plsc.subcore_barrier()

    return _prop


_prop4 = _make_prop(4)
_prop2 = _make_prop(2)


_DOT = dict(precision=lax.Precision.HIGHEST, preferred_element_type=jnp.float32)
_RB = 1024


def _pack_bf16_pairs(h):
    bi = lax.bitcast_convert_type(h, jnp.int32) + jnp.int32(0x8000)
    word = (bi[:, 64:] & jnp.int32(-65536)) | lax.shift_right_logical(
        bi[:, :64], 16)
    return word


def _tc1(xp, w1, dinv2):
    def body(x_ref, w_ref, dv_ref, o_ref):
        h = jnp.dot(x_ref[...] * dv_ref[...], w_ref[...], **_DOT)
        o_ref[0] = _pack_bf16_pairs(h)

    return pl.pallas_call(
        body,
        grid=(NP // _RB, 4),
        in_specs=[
            pl.BlockSpec((_RB, 256), lambda i, j: (i, 0)),
            pl.BlockSpec((256, 128), lambda i, j: (0, j)),
            pl.BlockSpec((_RB, 1), lambda i, j: (i, 0)),
        ],
        out_specs=pl.BlockSpec((1, _RB, 64), lambda i, j: (j, i, 0)),
        out_shape=jax.ShapeDtypeStruct((4, NP, 64), jnp.int32),
    )(xp, w1, dinv2)


def _tc2(agg1, dinv2, b1c, w2c):
    def body(a_ref, dv_ref, b_ref, w_ref, o_ref):
        dv = dv_ref[...][None]
        t = jnp.maximum(dv * a_ref[...] + b_ref[...][:, None, :], 0.0)
        acc = jnp.zeros((_RB, 128), jnp.float32)
        for c in range(4):
            acc = acc + jnp.dot(t[c], w_ref[0, c], **_DOT)
        o_ref[0] = _pack_bf16_pairs(dv_ref[...] * acc)

    return pl.pallas_call(
        body,
        grid=(NP // _RB, 2),
        in_specs=[
            pl.BlockSpec((4, _RB, 128), lambda i, o: (0, i, 0)),
            pl.BlockSpec((_RB, 1), lambda i, o: (i, 0)),
            pl.BlockSpec((4, 128), lambda i, o: (0, 0)),
            pl.BlockSpec((1, 4, 128, 128), lambda i, o: (o, 0, 0, 0)),
        ],
        out_specs=pl.BlockSpec((1, _RB, 64), lambda i, o: (o, i, 0)),
        out_shape=jax.ShapeDtypeStruct((2, NP, 64), jnp.int32),
    )(agg1, dinv2, b1c, w2c)


def _tc3(agg2, dinv2, b2c, l1c, bl1, l2, bl2, l3, bl3):
    def body(a_ref, dv_ref, b_ref, l1_ref, c1_ref, l2_ref, c2_ref,
             l3_ref, c3_ref, o_ref):
        dv = dv_ref[...][None]
        z = jnp.maximum(dv * a_ref[...] + b_ref[...][:, None, :], 0.0)
        m1 = jnp.dot(z[0], l1_ref[0], **_DOT) + jnp.dot(z[1], l1_ref[1], **_DOT)
        m1 = jnp.maximum(m1 + c1_ref[...], 0.0)
        m2 = jnp.maximum(jnp.dot(m1, l2_ref[...], **_DOT) + c2_ref[...], 0.0)
        lg = jnp.dot(m2, l3_ref[...], **_DOT) + c3_ref[...]
        lg = lg - jnp.max(lg, axis=-1, keepdims=True)
        e = jnp.exp(lg)
        o_ref[...] = e / jnp.sum(e, axis=-1, keepdims=True)

    return pl.pallas_call(
        body,
        grid=(NP // _RB,),
        in_specs=[
            pl.BlockSpec((2, _RB, 128), lambda i: (0, i, 0)),
            pl.BlockSpec((_RB, 1), lambda i: (i, 0)),
            pl.BlockSpec((2, 128), lambda i: (0, 0)),
            pl.BlockSpec((2, 128, 128), lambda i: (0, 0, 0)),
            pl.BlockSpec((1, 128), lambda i: (0, 0)),
            pl.BlockSpec((128, 64), lambda i: (0, 0)),
            pl.BlockSpec((1, 64), lambda i: (0, 0)),
            pl.BlockSpec((64, 40), lambda i: (0, 0)),
            pl.BlockSpec((1, 40), lambda i: (0, 0)),
        ],
        out_specs=pl.BlockSpec((_RB, 40), lambda i: (i, 0)),
        out_shape=jax.ShapeDtypeStruct((NP, 40), jnp.float32),
    )(agg2, dinv2, b2c, l1c, bl1, l2, bl2, l3, bl3)


def kernel(x, edge_index, edge_weights, W1, b1, W2, b2, L1, bl1, L2, bl2,
           L3, bl3):
    src = edge_index[0].astype(jnp.int32)
    dst = edge_index[1].astype(jnp.int32)
    ew = edge_weights.astype(jnp.float32)
    ci = jnp.asarray(_C, jnp.int32)

    pe = EP - N_EDGES - N_NODES
    loop = jnp.arange(N_NODES, dtype=jnp.int32)
    pad_ids = jnp.arange(pe, dtype=jnp.int32)
    src_p = jnp.concatenate([src, loop, pad_ids % N_NODES])
    dst_p = jnp.concatenate([dst, loop, N_NODES + pad_ids % (NP - N_NODES)])
    ew_p = jnp.concatenate([ew, jnp.ones((N_NODES,), jnp.float32),
                            jnp.zeros((pe,), jnp.float32)])
    dst2d = dst_p.reshape(16, NB, 128)
    dstb = dst_p.reshape(16, _NB2, _B)
    ew2d = ew_p.reshape(16, NB, 128)

    xp = jnp.pad(x.astype(jnp.float32), ((0, NP - N_NODES), (0, 0)))
    dinv = _dinv_kernel(dst2d, ew2d)
    dinv2 = dinv.reshape(NP, 1)

    b1c = b1.reshape(4, 128)[:, ci]
    b2c = b2.reshape(2, 128)[:, ci]
    w2c = W2.reshape(4, 128, 2, 128)[:, ci].transpose(2, 0, 1, 3)
    l1c = L1.reshape(2, 128, 128)[:, ci]

    h1b = _tc1(xp, W1, dinv2)
    agg1 = _prop4(src_p, dstb, ew_p, h1b.reshape(4 * NP, 64))
    h2b = _tc2(agg1, dinv2, b1c, w2c)
    agg2 = _prop2(src_p, dstb, ew_p, h2b.reshape(2 * NP, 64))
    out = _tc3(agg2, dinv2, b2c, l1c, bl1.reshape(1, 128),
               L2, bl2.reshape(1, 64), L3, bl3.reshape(1, 40))
    return out[:N_NODES]


# speedup vs baseline: 1.8765x; 1.0135x over previous
import functools

import jax
import jax.numpy as jnp
from jax import lax
from jax.experimental import pallas as pl
from jax.experimental.pallas import tpu as pltpu
from jax.experimental.pallas import tpu_sc as plsc

N_NODES = 10000
N_EDGES = 160000
NP = 10240
EP = 174080
E_T = EP // 16
_B = 64
_NB2 = E_T // _B
NB = E_T // 128
N_STRIPE = NP // 16

_C = tuple((16 * (p // 32) + (p % 16) + 64 * ((p % 32) // 16))
           for p in range(128))

_MESH = dict(core_axis_name="c", subcore_axis_name="s")


def _newton_rsqrt(v):
    i = lax.bitcast_convert_type(v, jnp.int32)
    i = jnp.int32(0x5F3759DF) - lax.shift_right_arithmetic(i, 1)
    y = lax.bitcast_convert_type(i, jnp.float32)
    for _ in range(3):
        y = y * (1.5 - 0.5 * v * y * y)
    return y


@functools.partial(
    pl.kernel,
    mesh=plsc.VectorSubcoreMesh(**_MESH),
    out_type=[jax.ShapeDtypeStruct((NP,), jnp.float32),
              jax.ShapeDtypeStruct((EP,), jnp.float32)],
    scratch_types=[
        pltpu.VMEM((NB, 128), jnp.int32),
        pltpu.VMEM((E_T,), jnp.float32),
        pltpu.VMEM((E_T,), jnp.int32),
        pltpu.VMEM((128,), jnp.float32),
        pltpu.VMEM((N_STRIPE,), jnp.float32),
        pltpu.VMEM_SHARED((NP,), jnp.float32),
        pltpu.SemaphoreType.DMA,
    ],
)
def _dinv_kernel(dst_hbm, ew_hbm, src_hbm, out_hbm, ew2_hbm,
                 dstv, ewv, srcv, dvb, nodev, deg_sh, sem):
    cid = lax.axis_index("c")
    sid = lax.axis_index("s")

    def _zero(i, _):
        nodev[pl.ds(i * 16, 16)] = jnp.zeros((16,), jnp.float32)
        return _
    lax.fori_loop(0, N_STRIPE // 16, _zero, None)
    pltpu.sync_copy(nodev, deg_sh.at[pl.ds(sid * N_STRIPE, N_STRIPE)])
    plsc.subcore_barrier()

    pltpu.sync_copy(dst_hbm.at[sid], dstv)
    pltpu.sync_copy(ew_hbm.at[pl.ds(sid * E_T, E_T)], ewv)
    pltpu.sync_copy(src_hbm.at[pl.ds(sid * E_T, E_T)], srcv)

    def _scat(j, _):
        pltpu.sync_copy(ewv.at[pl.ds(j * 128, 128)],
                        deg_sh.at[dstv.at[j]], add=True)
        return _
    lax.fori_loop(0, NB, _scat, None)
    plsc.subcore_barrier()

    l0 = sid * N_STRIPE
    pltpu.sync_copy(deg_sh.at[pl.ds(l0, N_STRIPE)], nodev)

    def _rs(i, _):
        v = jnp.maximum(nodev[pl.ds(i * 16, 16)], 1.0)
        nodev[pl.ds(i * 16, 16)] = _newton_rsqrt(v)
        return _
    lax.fori_loop(0, N_STRIPE // 16, _rs, None)

    @pl.when(cid == 0)
    def _():
        pltpu.sync_copy(nodev, out_hbm.at[pl.ds(l0, N_STRIPE)])
    pltpu.sync_copy(nodev, deg_sh.at[pl.ds(l0, N_STRIPE)])
    plsc.subcore_barrier()

    def _ew2(j, _):
        pltpu.sync_copy(deg_sh.at[srcv.at[pl.ds(j * 128, 128)]], dvb)
        for k in range(8):
            sl = pl.ds(j * 128 + k * 16, 16)
            ewv[sl] = ewv[sl] * dvb[pl.ds(k * 16, 16)]
        return _
    lax.fori_loop(0, E_T // 128, _ew2, None)
    pltpu.sync_copy(ewv, ew2_hbm.at[pl.ds(sid * E_T, E_T)])


def _make_prop(wc):
    cps = wc // 2

    @functools.partial(
        pl.kernel,
        mesh=plsc.VectorSubcoreMesh(**_MESH),
        out_type=jax.ShapeDtypeStruct((wc, NP, 128), jnp.float32),
        compiler_params=pltpu.CompilerParams(
            needs_layout_passes=False, use_tc_tiling_on_sc=False),
        scratch_types=[
            pltpu.VMEM((_B,), jnp.float32),
            pltpu.VMEM((_B,), jnp.float32),
            pltpu.VMEM((_NB2, _B), jnp.int32),
            pltpu.VMEM((E_T,), jnp.int32),
            pltpu.VMEM((_B, 64), jnp.int32),
            pltpu.VMEM((_B, 64), jnp.int32),
            pltpu.VMEM((_B, 128), jnp.float32),
            pltpu.VMEM((_B, 128), jnp.float32),
            pltpu.VMEM_SHARED((NP, 128), jnp.float32),
            pltpu.SemaphoreType.DMA,
            pltpu.SemaphoreType.DMA,
            pltpu.SemaphoreType.DMA,
            pltpu.SemaphoreType.DMA,
            pltpu.SemaphoreType.DMA,
            pltpu.SemaphoreType.DMA,
        ],
    )
    def _prop(src_hbm, dst_hbm, ew_hbm, tbl_hbm, out_hbm,
              ewb0, ewb1, dstv, idxv, rb0, rb1,
              rf0, rf1, agg_sh,
              semg0, semg1, sems0, sems1, seme0, seme1):
        cid = lax.axis_index("c")
        sid = lax.axis_index("s")
        e0 = sid * E_T
        pltpu.sync_copy(src_hbm.at[pl.ds(e0, E_T)], idxv)
        pltpu.sync_copy(dst_hbm.at[sid], dstv)

        rowsb = (rb0, rb1)
        rowsf = (rf0, rf1)
        ewb = (ewb0, ewb1)
        semg = (semg0, semg1)
        sems = (sems0, sems1)
        seme = (seme0, seme1)

        def _gather(b, buf):
            pltpu.async_copy(
                ew_hbm.at[pl.ds(e0 + b * _B, _B)], ewb[buf], seme[buf])
            pltpu.async_copy(
                tbl_hbm.at[idxv.at[pl.ds(b * _B, _B)]], rowsb[buf], semg[buf])

        def _gather_wait(b, buf):
            pltpu.make_async_copy(
                tbl_hbm.at[idxv.at[pl.ds(b * _B, _B)]], rowsb[buf], semg[buf]
            ).wait()

        def _scat(b, buf):
            pltpu.async_copy(
                rowsf[buf], agg_sh.at[dstv.at[b]], sems[buf], add=True)

        def _scat_wait(b, buf):
            pltpu.make_async_copy(
                rowsf[buf], agg_sh.at[dstv.at[b]], sems[buf]).wait()

        def _scale(b, gbuf, fbuf):
            pltpu.make_async_copy(
                ew_hbm.at[pl.ds(e0 + b * _B, _B)], ewb[gbuf], seme[gbuf]
            ).wait()
            mask = jnp.full((16,), -65536, jnp.int32)
            for g in range(_B // 16):
                wv = ewb[gbuf][pl.ds(g * 16, 16)]
                for e16 in range(16):
                    w = wv[e16]
                    r = g * 16 + e16
                    for k in range(4):
                        mi = rowsb[gbuf][r, pl.ds(k * 16, 16)]
                        lo = lax.bitcast_convert_type(
                            lax.shift_left(mi, 16), jnp.float32)
                        hi = lax.bitcast_convert_type(mi & mask, jnp.float32)
                        rowsf[fbuf][r, pl.ds(k * 32, 16)] = lo * w
                        rowsf[fbuf][r, pl.ds(k * 32 + 16, 16)] = hi * w

        for fci in range(cps):
            fcg = cid * cps + fci
            def _zb(j, _):
                for k in range(8):
                    rf0[j, pl.ds(k * 16, 16)] = jnp.zeros((16,), jnp.float32)
                return _
            lax.fori_loop(0, _B, _zb, None)
            for t in range(N_STRIPE // _B):
                pltpu.sync_copy(
                    rf0, agg_sh.at[pl.ds(sid * N_STRIPE + t * _B, _B)])
            delta = cid * cps * NP if fci == 0 else NP

            def _idx(i, _):
                idxv[pl.ds(i * 16, 16)] = idxv[pl.ds(i * 16, 16)] + delta
                return _
            lax.fori_loop(0, E_T // 16, _idx, None)
            plsc.subcore_barrier()

            _gather(0, 0)

            def _pair(b2, _):
                b = 2 * b2

                @pl.when(b2 > 0)
                def _():
                    _scat_wait(b - 1, 1)
                _gather(b + 1, 1)
                _gather_wait(b, 0)
                _scale(b, 0, 0)
                _scat(b, 0)

                _scat_wait(b, 0)

                @pl.when(b2 < _NB2 // 2 - 1)
                def _():
                    _gather(b + 2, 0)
                _gather_wait(b + 1, 1)
                _scale(b + 1, 1, 1)
                _scat(b + 1, 1)
                return _
            lax.fori_loop(0, _NB2 // 2, _pair, None)
            _scat_wait(_NB2 - 1, 1)
            plsc.subcore_barrier()
            pltpu.sync_copy(
                agg_sh.at[pl.ds(sid * N_STRIPE, N_STRIPE)],
                out_hbm.at[fcg, pl.ds(sid * N_STRIPE, N_STRIPE)])
            plsc.subcore_barrier()

    return _prop


_prop4 = _make_prop(4)
_prop2 = _make_prop(2)


_DOT = dict(precision=lax.Precision.HIGHEST, preferred_element_type=jnp.float32)
_RB = 1024


def _pack_bf16_pairs(h):
    bi = lax.bitcast_convert_type(h, jnp.int32) + jnp.int32(0x8000)
    word = (bi[:, 64:] & jnp.int32(-65536)) | lax.shift_right_logical(
        bi[:, :64], 16)
    return word


def _tc1(xp, w1):
    def body(x_ref, w_ref, o_ref):
        h = jnp.dot(x_ref[...], w_ref[...], **_DOT)
        o_ref[0] = _pack_bf16_pairs(h)

    return pl.pallas_call(
        body,
        grid=(NP // _RB, 4),
        in_specs=[
            pl.BlockSpec((_RB, 256), lambda i, j: (i, 0)),
            pl.BlockSpec((256, 128), lambda i, j: (0, j)),
        ],
        out_specs=pl.BlockSpec((1, _RB, 64), lambda i, j: (j, i, 0)),
        out_shape=jax.ShapeDtypeStruct((4, NP, 64), jnp.int32),
    )(xp, w1)


def _tc2(agg1, dinv2, b1c, w2c):
    def body(a_ref, dv_ref, b_ref, w_ref, o_ref):
        dv = dv_ref[...][None]
        t = jnp.maximum(dv * a_ref[...] + b_ref[...][:, None, :], 0.0)
        acc = jnp.zeros((_RB, 128), jnp.float32)
        for c in range(4):
            acc = acc + jnp.dot(t[c], w_ref[0, c], **_DOT)
        o_ref[0] = _pack_bf16_pairs(acc)

    return pl.pallas_call(
        body,
        grid=(NP // _RB, 2),
        in_specs=[
            pl.BlockSpec((4, _RB, 128), lambda i, o: (0, i, 0)),
            pl.BlockSpec((_RB, 1), lambda i, o: (i, 0)),
            pl.BlockSpec((4, 128), lambda i, o: (0, 0)),
            pl.BlockSpec((1, 4, 128, 128), lambda i, o: (o, 0, 0, 0)),
        ],
        out_specs=pl.BlockSpec((1, _RB, 64), lambda i, o: (o, i, 0)),
        out_shape=jax.ShapeDtypeStruct((2, NP, 64), jnp.int32),
    )(agg1, dinv2, b1c, w2c)


def _tc3(agg2, dinv2, b2c, l1c, bl1, l2, bl2, l3, bl3):
    def body(a_ref, dv_ref, b_ref, l1_ref, c1_ref, l2_ref, c2_ref,
             l3_ref, c3_ref, o_ref):
        dv = dv_ref[...][None]
        z = jnp.maximum(dv * a_ref[...] + b_ref[...][:, None, :], 0.0)
        m1 = jnp.dot(z[0], l1_ref[0], **_DOT) + jnp.dot(z[1], l1_ref[1], **_DOT)
        m1 = jnp.maximum(m1 + c1_ref[...], 0.0)
        m2 = jnp.maximum(jnp.dot(m1, l2_ref[...], **_DOT) + c2_ref[...], 0.0)
        lg = jnp.dot(m2, l3_ref[...], **_DOT) + c3_ref[...]
        lg = lg - jnp.max(lg, axis=-1, keepdims=True)
        e = jnp.exp(lg)
        o_ref[...] = e / jnp.sum(e, axis=-1, keepdims=True)

    return pl.pallas_call(
        body,
        grid=(NP // _RB,),
        in_specs=[
            pl.BlockSpec((2, _RB, 128), lambda i: (0, i, 0)),
            pl.BlockSpec((_RB, 1), lambda i: (i, 0)),
            pl.BlockSpec((2, 128), lambda i: (0, 0)),
            pl.BlockSpec((2, 128, 128), lambda i: (0, 0, 0)),
            pl.BlockSpec((1, 128), lambda i: (0, 0)),
            pl.BlockSpec((128, 64), lambda i: (0, 0)),
            pl.BlockSpec((1, 64), lambda i: (0, 0)),
            pl.BlockSpec((64, 40), lambda i: (0, 0)),
            pl.BlockSpec((1, 40), lambda i: (0, 0)),
        ],
        out_specs=pl.BlockSpec((_RB, 40), lambda i: (i, 0)),
        out_shape=jax.ShapeDtypeStruct((NP, 40), jnp.float32),
    )(agg2, dinv2, b2c, l1c, bl1, l2, bl2, l3, bl3)


def kernel(x, edge_index, edge_weights, W1, b1, W2, b2, L1, bl1, L2, bl2,
           L3, bl3):
    src = edge_index[0].astype(jnp.int32)
    dst = edge_index[1].astype(jnp.int32)
    ew = edge_weights.astype(jnp.float32)
    ci = jnp.asarray(_C, jnp.int32)

    pe = EP - N_EDGES - N_NODES
    loop = jnp.arange(N_NODES, dtype=jnp.int32)
    pad_ids = jnp.arange(pe, dtype=jnp.int32)
    src_p = jnp.concatenate([src, loop, pad_ids % N_NODES])
    dst_p = jnp.concatenate([dst, loop, N_NODES + pad_ids % (NP - N_NODES)])
    ew_p = jnp.concatenate([ew, jnp.ones((N_NODES,), jnp.float32),
                            jnp.zeros((pe,), jnp.float32)])
    dst2d = dst_p.reshape(16, NB, 128)
    dstb = dst_p.reshape(16, _NB2, _B)

    xp = jnp.pad(x.astype(jnp.float32), ((0, NP - N_NODES), (0, 0)))
    dinv, ew2 = _dinv_kernel(dst2d, ew_p, src_p)
    dinv2 = dinv.reshape(NP, 1)

    b1c = b1.reshape(4, 128)[:, ci]
    b2c = b2.reshape(2, 128)[:, ci]
    w2c = W2.reshape(4, 128, 2, 128)[:, ci].transpose(2, 0, 1, 3)
    l1c = L1.reshape(2, 128, 128)[:, ci]

    h1b = _tc1(xp, W1)
    agg1 = _prop4(src_p, dstb, ew2, h1b.reshape(4 * NP, 64))
    h2b = _tc2(agg1, dinv2, b1c, w2c)
    agg2 = _prop2(src_p, dstb, ew2, h2b.reshape(2 * NP, 64))
    out = _tc3(agg2, dinv2, b2c, l1c, bl1.reshape(1, 128),
               L2, bl2.reshape(1, 64), L3, bl3.reshape(1, 40))
    return out[:N_NODES]


# speedup vs baseline: 2.1615x; 1.1519x over previous
import functools

import jax
import jax.numpy as jnp
from jax import lax
from jax.experimental import pallas as pl
from jax.experimental.pallas import tpu as pltpu
from jax.experimental.pallas import tpu_sc as plsc

N_NODES = 10000
N_EDGES = 160000
NP = 10240
EP = 174080
E_T = EP // 16
_B = 64
_NB2 = E_T // _B
NB = E_T // 128
N_STRIPE = NP // 16

_C = tuple((16 * (p // 32) + (p % 16) + 64 * ((p % 32) // 16))
           for p in range(128))

_MESH = dict(core_axis_name="c", subcore_axis_name="s")


def _newton_rsqrt(v):
    i = lax.bitcast_convert_type(v, jnp.int32)
    i = jnp.int32(0x5F3759DF) - lax.shift_right_arithmetic(i, 1)
    y = lax.bitcast_convert_type(i, jnp.float32)
    for _ in range(3):
        y = y * (1.5 - 0.5 * v * y * y)
    return y


@functools.partial(
    pl.kernel,
    mesh=plsc.VectorSubcoreMesh(**_MESH),
    out_type=[jax.ShapeDtypeStruct((NP,), jnp.float32),
              jax.ShapeDtypeStruct((EP,), jnp.float32)],
    scratch_types=[
        pltpu.VMEM((NB, 128), jnp.int32),
        pltpu.VMEM((E_T,), jnp.float32),
        pltpu.VMEM((E_T,), jnp.int32),
        pltpu.VMEM((128,), jnp.float32),
        pltpu.VMEM((N_STRIPE,), jnp.float32),
        pltpu.VMEM_SHARED((NP,), jnp.float32),
        pltpu.SemaphoreType.DMA,
    ],
)
def _dinv_kernel(dst_hbm, ew_hbm, src_hbm, out_hbm, ew2_hbm,
                 dstv, ewv, srcv, dvb, nodev, deg_sh, sem):
    cid = lax.axis_index("c")
    sid = lax.axis_index("s")

    def _zero(i, _):
        nodev[pl.ds(i * 16, 16)] = jnp.zeros((16,), jnp.float32)
        return _
    lax.fori_loop(0, N_STRIPE // 16, _zero, None)
    pltpu.sync_copy(nodev, deg_sh.at[pl.ds(sid * N_STRIPE, N_STRIPE)])
    plsc.subcore_barrier()

    pltpu.sync_copy(dst_hbm.at[sid], dstv)
    pltpu.sync_copy(ew_hbm.at[pl.ds(sid * E_T, E_T)], ewv)
    pltpu.sync_copy(src_hbm.at[pl.ds(sid * E_T, E_T)], srcv)

    def _scat(j, _):
        pltpu.sync_copy(ewv.at[pl.ds(j * 128, 128)],
                        deg_sh.at[dstv.at[j]], add=True)
        return _
    lax.fori_loop(0, NB, _scat, None)
    plsc.subcore_barrier()

    l0 = sid * N_STRIPE
    pltpu.sync_copy(deg_sh.at[pl.ds(l0, N_STRIPE)], nodev)

    def _rs(i, _):
        v = jnp.maximum(nodev[pl.ds(i * 16, 16)], 1.0)
        nodev[pl.ds(i * 16, 16)] = _newton_rsqrt(v)
        return _
    lax.fori_loop(0, N_STRIPE // 16, _rs, None)

    @pl.when(cid == 0)
    def _():
        pltpu.sync_copy(nodev, out_hbm.at[pl.ds(l0, N_STRIPE)])
    pltpu.sync_copy(nodev, deg_sh.at[pl.ds(l0, N_STRIPE)])
    plsc.subcore_barrier()

    def _ew2(j, _):
        pltpu.sync_copy(deg_sh.at[srcv.at[pl.ds(j * 128, 128)]], dvb)
        for k in range(8):
            sl = pl.ds(j * 128 + k * 16, 16)
            ewv[sl] = ewv[sl] * dvb[pl.ds(k * 16, 16)]
        return _
    lax.fori_loop(0, E_T // 128, _ew2, None)
    pltpu.sync_copy(ewv, ew2_hbm.at[pl.ds(sid * E_T, E_T)])


def _make_prop(wc):
    cps = wc // 2

    @functools.partial(
        pl.kernel,
        mesh=plsc.VectorSubcoreMesh(**_MESH),
        out_type=jax.ShapeDtypeStruct((wc, NP, 128), jnp.float32),
        compiler_params=pltpu.CompilerParams(
            needs_layout_passes=False, use_tc_tiling_on_sc=False),
        scratch_types=[
            pltpu.VMEM((_B,), jnp.float32),
            pltpu.VMEM((_B,), jnp.float32),
            pltpu.VMEM((_NB2, _B), jnp.int32),
            pltpu.VMEM((E_T,), jnp.int32),
            pltpu.VMEM((_B, 64), jnp.int32),
            pltpu.VMEM((_B, 64), jnp.int32),
            pltpu.VMEM((_B, 128), jnp.float32),
            pltpu.VMEM((_B, 128), jnp.float32),
            pltpu.VMEM_SHARED((NP, 128), jnp.float32),
            pltpu.SemaphoreType.DMA,
            pltpu.SemaphoreType.DMA,
            pltpu.SemaphoreType.DMA,
            pltpu.SemaphoreType.DMA,
            pltpu.SemaphoreType.DMA,
            pltpu.SemaphoreType.DMA,
        ],
    )
    def _prop(src_hbm, dst_hbm, ew_hbm, tbl_hbm, out_hbm,
              ewb0, ewb1, dstv, idxv, rb0, rb1,
              rf0, rf1, agg_sh,
              semg0, semg1, sems0, sems1, seme0, seme1):
        cid = lax.axis_index("c")
        sid = lax.axis_index("s")
        e0 = sid * E_T
        pltpu.sync_copy(src_hbm.at[pl.ds(e0, E_T)], idxv)
        pltpu.sync_copy(dst_hbm.at[sid], dstv)

        rowsb = (rb0, rb1)
        rowsf = (rf0, rf1)
        ewb = (ewb0, ewb1)
        semg = (semg0, semg1)
        sems = (sems0, sems1)
        seme = (seme0, seme1)

        def _gather(b, buf):
            pltpu.async_copy(
                ew_hbm.at[pl.ds(e0 + b * _B, _B)], ewb[buf], seme[buf])
            pltpu.async_copy(
                tbl_hbm.at[idxv.at[pl.ds(b * _B, _B)]], rowsb[buf], semg[buf])

        def _gather_wait(b, buf):
            pltpu.make_async_copy(
                tbl_hbm.at[idxv.at[pl.ds(b * _B, _B)]], rowsb[buf], semg[buf]
            ).wait()

        def _scat(b, buf):
            pltpu.async_copy(
                rowsf[buf], agg_sh.at[dstv.at[b]], sems[buf], add=True)

        def _scat_wait(b, buf):
            pltpu.make_async_copy(
                rowsf[buf], agg_sh.at[dstv.at[b]], sems[buf]).wait()

        def _scale(b, gbuf, fbuf):
            pltpu.make_async_copy(
                ew_hbm.at[pl.ds(e0 + b * _B, _B)], ewb[gbuf], seme[gbuf]
            ).wait()
            mask = jnp.full((16,), -65536, jnp.int32)
            for g in range(_B // 16):
                wv = ewb[gbuf][pl.ds(g * 16, 16)]
                for e16 in range(16):
                    w = wv[e16]
                    r = g * 16 + e16
                    for k in range(4):
                        mi = rowsb[gbuf][r, pl.ds(k * 16, 16)]
                        lo = lax.bitcast_convert_type(
                            lax.shift_left(mi, 16), jnp.float32)
                        hi = lax.bitcast_convert_type(mi & mask, jnp.float32)
                        rowsf[fbuf][r, pl.ds(k * 32, 16)] = lo * w
                        rowsf[fbuf][r, pl.ds(k * 32 + 16, 16)] = hi * w

        for fci in range(cps):
            fcg = cid * cps + fci
            def _zb(j, _):
                for k in range(8):
                    rf0[j, pl.ds(k * 16, 16)] = jnp.zeros((16,), jnp.float32)
                return _
            lax.fori_loop(0, _B, _zb, None)
            for t in range(N_STRIPE // _B):
                pltpu.sync_copy(
                    rf0, agg_sh.at[pl.ds(sid * N_STRIPE + t * _B, _B)])
            delta = cid * cps * NP if fci == 0 else NP

            def _idx(i, _):
                idxv[pl.ds(i * 16, 16)] = idxv[pl.ds(i * 16, 16)] + delta
                return _
            lax.fori_loop(0, E_T // 16, _idx, None)
            plsc.subcore_barrier()

            _gather(0, 0)

            def _pair(b2, _):
                b = 2 * b2

                @pl.when(b2 > 0)
                def _():
                    _scat_wait(b - 2, 0)
                _gather(b + 1, 1)
                _gather_wait(b, 0)
                _scale(b, 0, 0)
                _scat(b, 0)

                @pl.when(b2 > 0)
                def _():
                    _scat_wait(b - 1, 1)

                @pl.when(b2 < _NB2 // 2 - 1)
                def _():
                    _gather(b + 2, 0)
                _gather_wait(b + 1, 1)
                _scale(b + 1, 1, 1)
                _scat(b + 1, 1)
                return _
            lax.fori_loop(0, _NB2 // 2, _pair, None)
            _scat_wait(_NB2 - 2, 0)
            _scat_wait(_NB2 - 1, 1)
            plsc.subcore_barrier()
            pltpu.sync_copy(
                agg_sh.at[pl.ds(sid * N_STRIPE, N_STRIPE)],
                out_hbm.at[fcg, pl.ds(sid * N_STRIPE, N_STRIPE)])
            plsc.subcore_barrier()

    return _prop


_prop4 = _make_prop(4)
_prop2 = _make_prop(2)


_DOT = dict(precision=lax.Precision.HIGHEST, preferred_element_type=jnp.float32)
_RB = 1024


def _pack_bf16_pairs(h):
    bi = lax.bitcast_convert_type(h, jnp.int32) + jnp.int32(0x8000)
    word = (bi[:, 64:] & jnp.int32(-65536)) | lax.shift_right_logical(
        bi[:, :64], 16)
    return word


def _tc1(xp, w1):
    def body(x_ref, w_ref, o_ref):
        h = jnp.dot(x_ref[...], w_ref[...], **_DOT)
        o_ref[0] = _pack_bf16_pairs(h)

    return pl.pallas_call(
        body,
        grid=(NP // _RB, 4),
        in_specs=[
            pl.BlockSpec((_RB, 256), lambda i, j: (i, 0)),
            pl.BlockSpec((256, 128), lambda i, j: (0, j)),
        ],
        out_specs=pl.BlockSpec((1, _RB, 64), lambda i, j: (j, i, 0)),
        out_shape=jax.ShapeDtypeStruct((4, NP, 64), jnp.int32),
    )(xp, w1)


def _tc2(agg1, dinv2, b1c, w2c):
    def body(a_ref, dv_ref, b_ref, w_ref, o_ref):
        dv = dv_ref[...][None]
        t = jnp.maximum(dv * a_ref[...] + b_ref[...][:, None, :], 0.0)
        acc = jnp.zeros((_RB, 128), jnp.float32)
        for c in range(4):
            acc = acc + jnp.dot(t[c], w_ref[0, c], **_DOT)
        o_ref[0] = _pack_bf16_pairs(acc)

    return pl.pallas_call(
        body,
        grid=(NP // _RB, 2),
        in_specs=[
            pl.BlockSpec((4, _RB, 128), lambda i, o: (0, i, 0)),
            pl.BlockSpec((_RB, 1), lambda i, o: (i, 0)),
            pl.BlockSpec((4, 128), lambda i, o: (0, 0)),
            pl.BlockSpec((1, 4, 128, 128), lambda i, o: (o, 0, 0, 0)),
        ],
        out_specs=pl.BlockSpec((1, _RB, 64), lambda i, o: (o, i, 0)),
        out_shape=jax.ShapeDtypeStruct((2, NP, 64), jnp.int32),
    )(agg1, dinv2, b1c, w2c)


def _tc3(agg2, dinv2, b2c, l1c, bl1, l2, bl2, l3, bl3):
    def body(a_ref, dv_ref, b_ref, l1_ref, c1_ref, l2_ref, c2_ref,
             l3_ref, c3_ref, o_ref):
        dv = dv_ref[...][None]
        z = jnp.maximum(dv * a_ref[...] + b_ref[...][:, None, :], 0.0)
        m1 = jnp.dot(z[0], l1_ref[0], **_DOT) + jnp.dot(z[1], l1_ref[1], **_DOT)
        m1 = jnp.maximum(m1 + c1_ref[...], 0.0)
        m2 = jnp.maximum(jnp.dot(m1, l2_ref[...], **_DOT) + c2_ref[...], 0.0)
        lg = jnp.dot(m2, l3_ref[...], **_DOT) + c3_ref[...]
        lg = lg - jnp.max(lg, axis=-1, keepdims=True)
        e = jnp.exp(lg)
        o_ref[...] = e / jnp.sum(e, axis=-1, keepdims=True)

    return pl.pallas_call(
        body,
        grid=(NP // _RB,),
        in_specs=[
            pl.BlockSpec((2, _RB, 128), lambda i: (0, i, 0)),
            pl.BlockSpec((_RB, 1), lambda i: (i, 0)),
            pl.BlockSpec((2, 128), lambda i: (0, 0)),
            pl.BlockSpec((2, 128, 128), lambda i: (0, 0, 0)),
            pl.BlockSpec((1, 128), lambda i: (0, 0)),
            pl.BlockSpec((128, 64), lambda i: (0, 0)),
            pl.BlockSpec((1, 64), lambda i: (0, 0)),
            pl.BlockSpec((64, 40), lambda i: (0, 0)),
            pl.BlockSpec((1, 40), lambda i: (0, 0)),
        ],
        out_specs=pl.BlockSpec((_RB, 40), lambda i: (i, 0)),
        out_shape=jax.ShapeDtypeStruct((NP, 40), jnp.float32),
    )(agg2, dinv2, b2c, l1c, bl1, l2, bl2, l3, bl3)


def kernel(x, edge_index, edge_weights, W1, b1, W2, b2, L1, bl1, L2, bl2,
           L3, bl3):
    src = edge_index[0].astype(jnp.int32)
    dst = edge_index[1].astype(jnp.int32)
    ew = edge_weights.astype(jnp.float32)
    ci = jnp.asarray(_C, jnp.int32)

    pe = EP - N_EDGES - N_NODES
    loop = jnp.arange(N_NODES, dtype=jnp.int32)
    pad_ids = jnp.arange(pe, dtype=jnp.int32)
    src_p = jnp.concatenate([src, loop, pad_ids % N_NODES])
    dst_p = jnp.concatenate([dst, loop, N_NODES + pad_ids % (NP - N_NODES)])
    ew_p = jnp.concatenate([ew, jnp.ones((N_NODES,), jnp.float32),
                            jnp.zeros((pe,), jnp.float32)])
    dst2d = dst_p.reshape(16, NB, 128)
    dstb = dst_p.reshape(16, _NB2, _B)

    xp = jnp.pad(x.astype(jnp.float32), ((0, NP - N_NODES), (0, 0)))
    dinv, ew2 = _dinv_kernel(dst2d, ew_p, src_p)
    dinv2 = dinv.reshape(NP, 1)

    b1c = b1.reshape(4, 128)[:, ci]
    b2c = b2.reshape(2, 128)[:, ci]
    w2c = W2.reshape(4, 128, 2, 128)[:, ci].transpose(2, 0, 1, 3)
    l1c = L1.reshape(2, 128, 128)[:, ci]

    h1b = _tc1(xp, W1)
    agg1 = _prop4(src_p, dstb, ew2, h1b.reshape(4 * NP, 64))
    h2b = _tc2(agg1, dinv2, b1c, w2c)
    agg2 = _prop2(src_p, dstb, ew2, h2b.reshape(2 * NP, 64))
    out = _tc3(agg2, dinv2, b2c, l1c, bl1.reshape(1, 128),
               L2, bl2.reshape(1, 64), L3, bl3.reshape(1, 40))
    return out[:N_NODES]
